# baseline (speedup 1.0000x reference)
import functools

import jax
import jax.numpy as jnp
from jax import lax
from jax.experimental import pallas as pl
from jax.experimental.pallas import tpu as pltpu
from jax.experimental.pallas import tpu_sc as plsc

import numpy as np

T = 2048
D = 1024
F = 4096
E = 16
CAP = 160
EC = E * CAP

NC, NS = 2, 16
NW = NC * NS
TPW = T // NW

ROWS_PAD = EC + NW
GATHER_SPREAD = EC // NW

_F_BLK = 2048
_NF = F // _F_BLK

_CUM_BLK = 128
_NCB = T // _CUM_BLK


def _gating_body(x_ref, wg_ref, s1_ref, s2_ref, gs1_ref, gs2_ref,
                 g1_ref, g2_ref):
    x = x_ref[...]
    wg = wg_ref[...]
    logits = jnp.dot(x, wg, preferred_element_type=jnp.float32)
    gates = jax.nn.softmax(logits, axis=-1)

    iota_e = lax.broadcasted_iota(jnp.int32, (T, E), 1)

    m1 = jnp.max(gates, axis=1, keepdims=True)
    idx1 = jnp.min(jnp.where(gates == m1, iota_e, E), axis=1, keepdims=True)
    mask1 = (iota_e == idx1).astype(jnp.float32)

    logits2 = jnp.where(mask1 > 0, logits - 1e9, logits)
    m2 = jnp.max(logits2, axis=1, keepdims=True)
    idx2 = jnp.min(jnp.where(logits2 == m2, iota_e, E), axis=1, keepdims=True)
    mask2 = (iota_e == idx2).astype(jnp.float32)

    br = lax.broadcasted_iota(jnp.int32, (_CUM_BLK, _CUM_BLK), 0)
    bc = lax.broadcasted_iota(jnp.int32, (_CUM_BLK, _CUM_BLK), 1)
    tri = (br >= bc).astype(jnp.float32)

    m12 = jnp.concatenate([mask1, mask2], axis=1)
    carry = jnp.zeros((1, 2 * E), jnp.float32)
    blocks = []
    for b in range(_NCB):
        blk = lax.slice(m12, (b * _CUM_BLK, 0), ((b + 1) * _CUM_BLK, 2 * E))
        part = jnp.dot(tri, blk, preferred_element_type=jnp.float32)
        blocks.append(part + carry)
        carry = carry + jnp.sum(blk, axis=0, keepdims=True)
    cum = jnp.concatenate(blocks, axis=0)

    cnt1 = jnp.sum(mask1, axis=0, keepdims=True)
    locations1 = cum[:, :E] - 1.0
    locations2 = cum[:, E:] - 1.0 + cnt1

    mask1k = mask1 * (locations1 < CAP).astype(jnp.float32)
    mask2k = mask2 * (locations2 < CAP).astype(jnp.float32)

    loc1 = jnp.sum(locations1 * mask1k, axis=1, keepdims=True)
    loc2 = jnp.sum(locations2 * mask2k, axis=1, keepdims=True)

    gate1 = jnp.sum(gates * mask1k, axis=1, keepdims=True)
    gate2 = jnp.sum(gates * mask2k, axis=1, keepdims=True)
    denom = gate1 + gate2
    denom = jnp.where(denom < jnp.finfo(jnp.float32).eps, 1.0, denom)
    g1 = gate1 / denom
    g2 = gate2 / denom

    kept1 = jnp.sum(mask1k, axis=1, keepdims=True) > 0
    kept2 = jnp.sum(mask2k, axis=1, keepdims=True) > 0

    wid_col = lax.broadcasted_iota(jnp.int32, (T, 1), 0) // TPW
    trash = EC + wid_col
    spread = wid_col * GATHER_SPREAD

    slot1 = idx1 * CAP + loc1.astype(jnp.int32)
    slot2 = idx2 * CAP + loc2.astype(jnp.int32)
    s1_ref[...] = jnp.where(kept1, slot1, trash)
    s2_ref[...] = jnp.where(kept2, slot2, trash)
    gs1_ref[...] = jnp.where(kept1, slot1, spread)
    gs2_ref[...] = jnp.where(kept2, slot2, spread)
    g1_ref[...] = jnp.broadcast_to(jnp.where(kept1, g1, 0.0), (T, E))
    g2_ref[...] = jnp.broadcast_to(jnp.where(kept2, g2, 0.0), (T, E))


def _gating_call(x, wg):
    i32 = jnp.int32
    f32 = jnp.float32
    return pl.pallas_call(
        _gating_body,
        out_shape=(
            jax.ShapeDtypeStruct((T, 1), i32),
            jax.ShapeDtypeStruct((T, 1), i32),
            jax.ShapeDtypeStruct((T, 1), i32),
            jax.ShapeDtypeStruct((T, 1), i32),
            jax.ShapeDtypeStruct((T, E), f32),
            jax.ShapeDtypeStruct((T, E), f32),
        ),
    )(x, wg)


def _dispatch_body(x_hbm, s1_hbm, s2_hbm, out_hbm, s1v, s2v, xv, sem_in,
                   sem_s1, sem_s2):
    wid = lax.axis_index("s") * NC + lax.axis_index("c")
    base = wid * TPW
    c0 = pltpu.async_copy(s1_hbm.at[pl.ds(base, TPW)], s1v, sem_in)
    c1 = pltpu.async_copy(s2_hbm.at[pl.ds(base, TPW)], s2v, sem_in)
    c2 = pltpu.async_copy(x_hbm.at[pl.ds(base, TPW)], xv, sem_in)
    c0.wait()
    c1.wait()
    c2.wait()
    w1 = pltpu.async_copy(xv, out_hbm.at[s1v], sem_s1)
    w2 = pltpu.async_copy(xv, out_hbm.at[s2v], sem_s2)
    w1.wait()
    w2.wait()


def _dispatch_call(x, s1, s2):
    mesh = plsc.VectorSubcoreMesh(core_axis_name="c", subcore_axis_name="s")
    return pl.kernel(
        _dispatch_body,
        out_type=jax.ShapeDtypeStruct((ROWS_PAD, D), jnp.float32),
        mesh=mesh,
        scratch_types=[
            pltpu.VMEM((TPW,), jnp.int32),
            pltpu.VMEM((TPW,), jnp.int32),
            pltpu.VMEM((TPW, D), jnp.float32),
            pltpu.SemaphoreType.DMA,
            pltpu.SemaphoreType.DMA,
            pltpu.SemaphoreType.DMA,
        ],
    )(x, s1, s2)


def _ffn_body(disp_ref, w1_ref, w2_ref, out_ref):
    f = pl.program_id(1)
    h = jnp.maximum(
        jnp.dot(disp_ref[...], w1_ref[0], preferred_element_type=jnp.float32),
        0.0)
    part = jnp.dot(h, w2_ref[0], preferred_element_type=jnp.float32)

    @pl.when(f == 0)
    def _():
        out_ref[...] = part

    @pl.when(f > 0)
    def _():
        out_ref[...] = out_ref[...] + part


def _ffn_call(disp, w1, w2):
    return pl.pallas_call(
        _ffn_body,
        grid=(E, _NF),
        in_specs=[
            pl.BlockSpec((CAP, D), lambda e, f: (e, 0)),
            pl.BlockSpec((1, D, _F_BLK), lambda e, f: (e, 0, f)),
            pl.BlockSpec((1, _F_BLK, D), lambda e, f: (e, f, 0)),
        ],
        out_specs=pl.BlockSpec((CAP, D), lambda e, f: (e, 0)),
        out_shape=jax.ShapeDtypeStruct((EC, D), jnp.float32),
        compiler_params=pltpu.CompilerParams(
            dimension_semantics=("parallel", "arbitrary")),
    )(disp, w1, w2)


_CH = 16
_NCH = TPW // _CH


def _combine_body(eo_hbm, gs1_hbm, gs2_hbm, g1_hbm, g2_hbm, out_hbm,
                  i1v, i2v, g1v, g2v, v1a, v2a, v1b, v2b, ova, ovb,
                  sem_g, sem_o):
    wid = lax.axis_index("s") * NC + lax.axis_index("c")
    base = wid * TPW
    ci0 = pltpu.async_copy(gs1_hbm.at[wid], i1v, sem_g)
    ci1 = pltpu.async_copy(gs2_hbm.at[wid], i2v, sem_g)
    cg0 = pltpu.async_copy(g1_hbm.at[pl.ds(base, TPW)], g1v, sem_g)
    cg1 = pltpu.async_copy(g2_hbm.at[pl.ds(base, TPW)], g2v, sem_g)
    ci0.wait()
    ci1.wait()
    cg0.wait()
    cg1.wait()

    vbufs = ((v1a, v2a), (v1b, v2b))
    obufs = (ova, ovb)

    def start(c):
        v1, v2 = vbufs[c % 2]
        a = pltpu.async_copy(eo_hbm.at[i1v.at[c]], v1, sem_g)
        b = pltpu.async_copy(eo_hbm.at[i2v.at[c]], v2, sem_g)
        return a, b

    pend = start(0)
    owrites = [None, None]
    for c in range(_NCH):
        v1, v2 = vbufs[c % 2]
        ov = obufs[c % 2]
        pend[0].wait()
        pend[1].wait()
        if c + 1 < _NCH:
            pend = start(c + 1)
        if owrites[c % 2] is not None:
            owrites[c % 2].wait()

        def row(r, _):
            gv1 = g1v[c * _CH + r]
            gv2 = g2v[c * _CH + r]
            for j in range(D // 16):
                a = v1[r, pl.ds(j * 16, 16)]
                b = v2[r, pl.ds(j * 16, 16)]
                ov[r, pl.ds(j * 16, 16)] = (
                    jnp.where(gv1 > 0, gv1 * a, 0.0)
                    + jnp.where(gv2 > 0, gv2 * b, 0.0))
            return _

        lax.fori_loop(0, _CH, row, 0)
        owrites[c % 2] = pltpu.async_copy(
            ov, out_hbm.at[pl.ds(base + c * _CH, _CH)], sem_o)
    owrites[0].wait()
    owrites[1].wait()


def _combine_call(eo, gs1, gs2, g1, g2):
    mesh = plsc.VectorSubcoreMesh(core_axis_name="c", subcore_axis_name="s")
    f32 = jnp.float32
    return pl.kernel(
        _combine_body,
        out_type=jax.ShapeDtypeStruct((T, D), f32),
        mesh=mesh,
        scratch_types=[
            pltpu.VMEM((_NCH, _CH), jnp.int32),
            pltpu.VMEM((_NCH, _CH), jnp.int32),
            pltpu.VMEM((TPW, E), f32),
            pltpu.VMEM((TPW, E), f32),
            pltpu.VMEM((_CH, D), f32),
            pltpu.VMEM((_CH, D), f32),
            pltpu.VMEM((_CH, D), f32),
            pltpu.VMEM((_CH, D), f32),
            pltpu.VMEM((_CH, D), f32),
            pltpu.VMEM((_CH, D), f32),
            pltpu.SemaphoreType.DMA,
            pltpu.SemaphoreType.DMA,
        ],
    )(eo, gs1.reshape(NW, _NCH, _CH), gs2.reshape(NW, _NCH, _CH), g1, g2)


def kernel(x, Wg, W1, W2):
    s1, s2, gs1, gs2, g1, g2 = _gating_call(x, Wg)
    s1 = s1.reshape(T)
    s2 = s2.reshape(T)
    gs1 = gs1.reshape(T)
    gs2 = gs2.reshape(T)
    disp = _dispatch_call(x, s1, s2)
    eo = _ffn_call(disp, W1, W2)
    return _combine_call(eo, gs1, gs2, g1, g2)


# speedup vs baseline: 1.0090x; 1.0090x over previous
import functools

import jax
import jax.numpy as jnp
from jax import lax
from jax.experimental import pallas as pl
from jax.experimental.pallas import tpu as pltpu
from jax.experimental.pallas import tpu_sc as plsc

import numpy as np

T = 2048
D = 1024
F = 4096
E = 16
CAP = 160
EC = E * CAP

NC, NS = 2, 16
NW = NC * NS
TPW = T // NW

ROWS_PAD = EC + NW
GATHER_SPREAD = EC // NW

_F_BLK = 1024
_NF = F // _F_BLK

_CUM_BLK = 128
_NCB = T // _CUM_BLK


def _gating_body(x_ref, wg_ref, s1_ref, s2_ref, gs1_ref, gs2_ref,
                 g1_ref, g2_ref):
    x = x_ref[...]
    wg = wg_ref[...]
    logits = jnp.dot(x, wg, preferred_element_type=jnp.float32)
    gates = jax.nn.softmax(logits, axis=-1)

    iota_e = lax.broadcasted_iota(jnp.int32, (T, E), 1)

    m1 = jnp.max(gates, axis=1, keepdims=True)
    idx1 = jnp.min(jnp.where(gates == m1, iota_e, E), axis=1, keepdims=True)
    mask1 = (iota_e == idx1).astype(jnp.float32)

    logits2 = jnp.where(mask1 > 0, logits - 1e9, logits)
    m2 = jnp.max(logits2, axis=1, keepdims=True)
    idx2 = jnp.min(jnp.where(logits2 == m2, iota_e, E), axis=1, keepdims=True)
    mask2 = (iota_e == idx2).astype(jnp.float32)

    br = lax.broadcasted_iota(jnp.int32, (_CUM_BLK, _CUM_BLK), 0)
    bc = lax.broadcasted_iota(jnp.int32, (_CUM_BLK, _CUM_BLK), 1)
    tri = (br >= bc).astype(jnp.float32)

    m12 = jnp.concatenate([mask1, mask2], axis=1)
    carry = jnp.zeros((1, 2 * E), jnp.float32)
    blocks = []
    for b in range(_NCB):
        blk = lax.slice(m12, (b * _CUM_BLK, 0), ((b + 1) * _CUM_BLK, 2 * E))
        part = jnp.dot(tri, blk, preferred_element_type=jnp.float32)
        blocks.append(part + carry)
        carry = carry + jnp.sum(blk, axis=0, keepdims=True)
    cum = jnp.concatenate(blocks, axis=0)

    cnt1 = jnp.sum(mask1, axis=0, keepdims=True)
    locations1 = cum[:, :E] - 1.0
    locations2 = cum[:, E:] - 1.0 + cnt1

    mask1k = mask1 * (locations1 < CAP).astype(jnp.float32)
    mask2k = mask2 * (locations2 < CAP).astype(jnp.float32)

    loc1 = jnp.sum(locations1 * mask1k, axis=1, keepdims=True)
    loc2 = jnp.sum(locations2 * mask2k, axis=1, keepdims=True)

    gate1 = jnp.sum(gates * mask1k, axis=1, keepdims=True)
    gate2 = jnp.sum(gates * mask2k, axis=1, keepdims=True)
    denom = gate1 + gate2
    denom = jnp.where(denom < jnp.finfo(jnp.float32).eps, 1.0, denom)
    g1 = gate1 / denom
    g2 = gate2 / denom

    kept1 = jnp.sum(mask1k, axis=1, keepdims=True) > 0
    kept2 = jnp.sum(mask2k, axis=1, keepdims=True) > 0

    wid_col = lax.broadcasted_iota(jnp.int32, (T, 1), 0) // TPW
    trash = EC + wid_col
    spread = wid_col * GATHER_SPREAD

    slot1 = idx1 * CAP + loc1.astype(jnp.int32)
    slot2 = idx2 * CAP + loc2.astype(jnp.int32)
    s1_ref[...] = jnp.where(kept1, slot1, trash)
    s2_ref[...] = jnp.where(kept2, slot2, trash)
    gs1_ref[...] = jnp.where(kept1, slot1, spread)
    gs2_ref[...] = jnp.where(kept2, slot2, spread)
    g1_ref[...] = jnp.broadcast_to(jnp.where(kept1, g1, 0.0), (T, E))
    g2_ref[...] = jnp.broadcast_to(jnp.where(kept2, g2, 0.0), (T, E))


def _gating_call(x, wg):
    i32 = jnp.int32
    f32 = jnp.float32
    return pl.pallas_call(
        _gating_body,
        out_shape=(
            jax.ShapeDtypeStruct((T, 1), i32),
            jax.ShapeDtypeStruct((T, 1), i32),
            jax.ShapeDtypeStruct((T, 1), i32),
            jax.ShapeDtypeStruct((T, 1), i32),
            jax.ShapeDtypeStruct((T, E), f32),
            jax.ShapeDtypeStruct((T, E), f32),
        ),
    )(x, wg)


def _dispatch_body(x_hbm, s1_hbm, s2_hbm, out_hbm, s1v, s2v, xv, sem_in,
                   sem_s1, sem_s2):
    wid = lax.axis_index("s") * NC + lax.axis_index("c")
    base = wid * TPW
    c0 = pltpu.async_copy(s1_hbm.at[pl.ds(base, TPW)], s1v, sem_in)
    c1 = pltpu.async_copy(s2_hbm.at[pl.ds(base, TPW)], s2v, sem_in)
    c2 = pltpu.async_copy(x_hbm.at[pl.ds(base, TPW)], xv, sem_in)
    c0.wait()
    c1.wait()
    c2.wait()
    w1 = pltpu.async_copy(xv, out_hbm.at[s1v], sem_s1)
    w2 = pltpu.async_copy(xv, out_hbm.at[s2v], sem_s2)
    w1.wait()
    w2.wait()


def _dispatch_call(x, s1, s2):
    mesh = plsc.VectorSubcoreMesh(core_axis_name="c", subcore_axis_name="s")
    return pl.kernel(
        _dispatch_body,
        out_type=jax.ShapeDtypeStruct((ROWS_PAD, D), jnp.float32),
        mesh=mesh,
        scratch_types=[
            pltpu.VMEM((TPW,), jnp.int32),
            pltpu.VMEM((TPW,), jnp.int32),
            pltpu.VMEM((TPW, D), jnp.float32),
            pltpu.SemaphoreType.DMA,
            pltpu.SemaphoreType.DMA,
            pltpu.SemaphoreType.DMA,
        ],
    )(x, s1, s2)


def _ffn_body(disp_ref, w1_ref, w2_ref, out_ref):
    f = pl.program_id(1)
    h = jnp.maximum(
        jnp.dot(disp_ref[...].astype(jnp.bfloat16),
                w1_ref[0].astype(jnp.bfloat16),
                preferred_element_type=jnp.float32),
        0.0)
    part = jnp.dot(h.astype(jnp.bfloat16),
                   w2_ref[0].astype(jnp.bfloat16),
                   preferred_element_type=jnp.float32)

    @pl.when(f == 0)
    def _():
        out_ref[...] = part

    @pl.when(f > 0)
    def _():
        out_ref[...] = out_ref[...] + part


def _ffn_call(disp, w1, w2):
    return pl.pallas_call(
        _ffn_body,
        grid=(E, _NF),
        in_specs=[
            pl.BlockSpec((CAP, D), lambda e, f: (e, 0)),
            pl.BlockSpec((1, D, _F_BLK), lambda e, f: (e, 0, f)),
            pl.BlockSpec((1, _F_BLK, D), lambda e, f: (e, f, 0)),
        ],
        out_specs=pl.BlockSpec((CAP, D), lambda e, f: (e, 0)),
        out_shape=jax.ShapeDtypeStruct((EC, D), jnp.float32),
        compiler_params=pltpu.CompilerParams(
            dimension_semantics=("parallel", "arbitrary")),
    )(disp, w1, w2)


_CH = 16
_NCH = TPW // _CH


def _combine_body(eo_hbm, gs1_hbm, gs2_hbm, g1_hbm, g2_hbm, out_hbm,
                  i1v, i2v, g1v, g2v, v1a, v2a, v1b, v2b, ova, ovb,
                  sem_g, sem_o):
    wid = lax.axis_index("s") * NC + lax.axis_index("c")
    base = wid * TPW
    ci0 = pltpu.async_copy(gs1_hbm.at[wid], i1v, sem_g)
    ci1 = pltpu.async_copy(gs2_hbm.at[wid], i2v, sem_g)
    cg0 = pltpu.async_copy(g1_hbm.at[pl.ds(base, TPW)], g1v, sem_g)
    cg1 = pltpu.async_copy(g2_hbm.at[pl.ds(base, TPW)], g2v, sem_g)
    ci0.wait()
    ci1.wait()
    cg0.wait()
    cg1.wait()

    vbufs = ((v1a, v2a), (v1b, v2b))
    obufs = (ova, ovb)

    def start(c):
        v1, v2 = vbufs[c % 2]
        a = pltpu.async_copy(eo_hbm.at[i1v.at[c]], v1, sem_g)
        b = pltpu.async_copy(eo_hbm.at[i2v.at[c]], v2, sem_g)
        return a, b

    pend = start(0)
    owrites = [None, None]
    for c in range(_NCH):
        v1, v2 = vbufs[c % 2]
        ov = obufs[c % 2]
        pend[0].wait()
        pend[1].wait()
        if c + 1 < _NCH:
            pend = start(c + 1)
        if owrites[c % 2] is not None:
            owrites[c % 2].wait()

        def row(r, _):
            gv1 = g1v[c * _CH + r]
            gv2 = g2v[c * _CH + r]
            for j in range(D // 16):
                a = v1[r, pl.ds(j * 16, 16)]
                b = v2[r, pl.ds(j * 16, 16)]
                ov[r, pl.ds(j * 16, 16)] = (
                    jnp.where(gv1 > 0, gv1 * a, 0.0)
                    + jnp.where(gv2 > 0, gv2 * b, 0.0))
            return _

        lax.fori_loop(0, _CH, row, 0)
        owrites[c % 2] = pltpu.async_copy(
            ov, out_hbm.at[pl.ds(base + c * _CH, _CH)], sem_o)
    owrites[0].wait()
    owrites[1].wait()


def _combine_call(eo, gs1, gs2, g1, g2):
    mesh = plsc.VectorSubcoreMesh(core_axis_name="c", subcore_axis_name="s")
    f32 = jnp.float32
    return pl.kernel(
        _combine_body,
        out_type=jax.ShapeDtypeStruct((T, D), f32),
        mesh=mesh,
        scratch_types=[
            pltpu.VMEM((_NCH, _CH), jnp.int32),
            pltpu.VMEM((_NCH, _CH), jnp.int32),
            pltpu.VMEM((TPW, E), f32),
            pltpu.VMEM((TPW, E), f32),
            pltpu.VMEM((_CH, D), f32),
            pltpu.VMEM((_CH, D), f32),
            pltpu.VMEM((_CH, D), f32),
            pltpu.VMEM((_CH, D), f32),
            pltpu.VMEM((_CH, D), f32),
            pltpu.VMEM((_CH, D), f32),
            pltpu.SemaphoreType.DMA,
            pltpu.SemaphoreType.DMA,
        ],
    )(eo, gs1.reshape(NW, _NCH, _CH), gs2.reshape(NW, _NCH, _CH), g1, g2)


def kernel(x, Wg, W1, W2):
    s1, s2, gs1, gs2, g1, g2 = _gating_call(x, Wg)
    s1 = s1.reshape(T)
    s2 = s2.reshape(T)
    gs1 = gs1.reshape(T)
    gs2 = gs2.reshape(T)
    disp = _dispatch_call(x, s1, s2)
    eo = _ffn_call(disp, W1, W2)
    return _combine_call(eo, gs1, gs2, g1, g2)


# speedup vs baseline: 1.0106x; 1.0016x over previous
import functools

import jax
import jax.numpy as jnp
from jax import lax
from jax.experimental import pallas as pl
from jax.experimental.pallas import tpu as pltpu
from jax.experimental.pallas import tpu_sc as plsc

import numpy as np

T = 2048
D = 1024
F = 4096
E = 16
CAP = 160
EC = E * CAP

NC, NS = 2, 16
NW = NC * NS
TPW = T // NW

ROWS_PAD = EC + NW
GATHER_SPREAD = EC // NW

_F_BLK = 1024
_NF = F // _F_BLK

_CUM_BLK = 128
_NCB = T // _CUM_BLK


def _gating_body(x_ref, wg_ref, s1_ref, s2_ref, gs1_ref, gs2_ref,
                 g1_ref, g2_ref):
    x = x_ref[...]
    wg = wg_ref[...]
    logits = jnp.dot(x, wg, preferred_element_type=jnp.float32)
    gates = jax.nn.softmax(logits, axis=-1)

    iota_e = lax.broadcasted_iota(jnp.int32, (T, E), 1)

    m1 = jnp.max(gates, axis=1, keepdims=True)
    idx1 = jnp.min(jnp.where(gates == m1, iota_e, E), axis=1, keepdims=True)
    mask1 = (iota_e == idx1).astype(jnp.float32)

    logits2 = jnp.where(mask1 > 0, logits - 1e9, logits)
    m2 = jnp.max(logits2, axis=1, keepdims=True)
    idx2 = jnp.min(jnp.where(logits2 == m2, iota_e, E), axis=1, keepdims=True)
    mask2 = (iota_e == idx2).astype(jnp.float32)

    br = lax.broadcasted_iota(jnp.int32, (_CUM_BLK, _CUM_BLK), 0)
    bc = lax.broadcasted_iota(jnp.int32, (_CUM_BLK, _CUM_BLK), 1)
    tri = (br >= bc).astype(jnp.float32)

    m12 = jnp.concatenate([mask1, mask2], axis=1)
    carry = jnp.zeros((1, 2 * E), jnp.float32)
    blocks = []
    for b in range(_NCB):
        blk = lax.slice(m12, (b * _CUM_BLK, 0), ((b + 1) * _CUM_BLK, 2 * E))
        part = jnp.dot(tri, blk, preferred_element_type=jnp.float32)
        blocks.append(part + carry)
        carry = carry + jnp.sum(blk, axis=0, keepdims=True)
    cum = jnp.concatenate(blocks, axis=0)

    cnt1 = jnp.sum(mask1, axis=0, keepdims=True)
    locations1 = cum[:, :E] - 1.0
    locations2 = cum[:, E:] - 1.0 + cnt1

    mask1k = mask1 * (locations1 < CAP).astype(jnp.float32)
    mask2k = mask2 * (locations2 < CAP).astype(jnp.float32)

    loc1 = jnp.sum(locations1 * mask1k, axis=1, keepdims=True)
    loc2 = jnp.sum(locations2 * mask2k, axis=1, keepdims=True)

    gate1 = jnp.sum(gates * mask1k, axis=1, keepdims=True)
    gate2 = jnp.sum(gates * mask2k, axis=1, keepdims=True)
    denom = gate1 + gate2
    denom = jnp.where(denom < jnp.finfo(jnp.float32).eps, 1.0, denom)
    g1 = gate1 / denom
    g2 = gate2 / denom

    kept1 = jnp.sum(mask1k, axis=1, keepdims=True) > 0
    kept2 = jnp.sum(mask2k, axis=1, keepdims=True) > 0

    wid_col = lax.broadcasted_iota(jnp.int32, (T, 1), 0) // TPW
    trash = EC + wid_col
    spread = wid_col * GATHER_SPREAD

    slot1 = idx1 * CAP + loc1.astype(jnp.int32)
    slot2 = idx2 * CAP + loc2.astype(jnp.int32)
    s1_ref[...] = jnp.where(kept1, slot1, trash)
    s2_ref[...] = jnp.where(kept2, slot2, trash)
    gs1_ref[...] = jnp.where(kept1, slot1, spread)
    gs2_ref[...] = jnp.where(kept2, slot2, spread)
    g1_ref[...] = jnp.broadcast_to(jnp.where(kept1, g1, 0.0), (T, E))
    g2_ref[...] = jnp.broadcast_to(jnp.where(kept2, g2, 0.0), (T, E))


def _gating_call(x, wg):
    i32 = jnp.int32
    f32 = jnp.float32
    return pl.pallas_call(
        _gating_body,
        out_shape=(
            jax.ShapeDtypeStruct((T, 1), i32),
            jax.ShapeDtypeStruct((T, 1), i32),
            jax.ShapeDtypeStruct((T, 1), i32),
            jax.ShapeDtypeStruct((T, 1), i32),
            jax.ShapeDtypeStruct((T, E), f32),
            jax.ShapeDtypeStruct((T, E), f32),
        ),
    )(x, wg)


def _dispatch_body(x_hbm, s1_hbm, s2_hbm, out_hbm, s1v, s2v, xv, sem_in,
                   sem_s1, sem_s2):
    wid = lax.axis_index("s") * NC + lax.axis_index("c")
    base = wid * TPW
    c0 = pltpu.async_copy(s1_hbm.at[pl.ds(base, TPW)], s1v, sem_in)
    c1 = pltpu.async_copy(s2_hbm.at[pl.ds(base, TPW)], s2v, sem_in)
    c2 = pltpu.async_copy(x_hbm.at[pl.ds(base, TPW)], xv, sem_in)
    c0.wait()
    c1.wait()
    c2.wait()
    w1 = pltpu.async_copy(xv, out_hbm.at[s1v], sem_s1)
    w2 = pltpu.async_copy(xv, out_hbm.at[s2v], sem_s2)
    w1.wait()
    w2.wait()


def _dispatch_call(x, s1, s2):
    mesh = plsc.VectorSubcoreMesh(core_axis_name="c", subcore_axis_name="s")
    return pl.kernel(
        _dispatch_body,
        out_type=jax.ShapeDtypeStruct((ROWS_PAD, D), jnp.float32),
        mesh=mesh,
        scratch_types=[
            pltpu.VMEM((TPW,), jnp.int32),
            pltpu.VMEM((TPW,), jnp.int32),
            pltpu.VMEM((TPW, D), jnp.float32),
            pltpu.SemaphoreType.DMA,
            pltpu.SemaphoreType.DMA,
            pltpu.SemaphoreType.DMA,
        ],
    )(x, s1, s2)


def _ffn_body(disp_ref, w1_ref, w2_ref, out_ref):
    f = pl.program_id(1)
    h = jnp.maximum(
        jnp.dot(disp_ref[...], w1_ref[0], preferred_element_type=jnp.float32),
        0.0)
    part = jnp.dot(h, w2_ref[0], preferred_element_type=jnp.float32)

    @pl.when(f == 0)
    def _():
        out_ref[...] = part

    @pl.when((f > 0) & (f < _NF - 1))
    def _():
        out_ref[...] = out_ref[...] + part

    @pl.when(f == _NF - 1)
    def _():
        acc = out_ref[...] + part
        acc = jnp.clip(acc, -3e38, 3e38)
        out_ref[...] = jnp.where(acc == acc, acc, 0.0)


def _ffn_call(disp, w1, w2):
    return pl.pallas_call(
        _ffn_body,
        grid=(E, _NF),
        in_specs=[
            pl.BlockSpec((CAP, D), lambda e, f: (e, 0)),
            pl.BlockSpec((1, D, _F_BLK), lambda e, f: (e, 0, f)),
            pl.BlockSpec((1, _F_BLK, D), lambda e, f: (e, f, 0)),
        ],
        out_specs=pl.BlockSpec((CAP, D), lambda e, f: (e, 0)),
        out_shape=jax.ShapeDtypeStruct((EC, D), jnp.float32),
        compiler_params=pltpu.CompilerParams(
            dimension_semantics=("parallel", "arbitrary")),
    )(disp, w1, w2)


_CH = 16
_NCH = TPW // _CH


def _combine_body(eo_hbm, gs1_hbm, gs2_hbm, g1_hbm, g2_hbm, out_hbm,
                  i1v, i2v, g1v, g2v, v1a, v2a, v1b, v2b, ova, ovb,
                  sem_g, sem_o):
    wid = lax.axis_index("s") * NC + lax.axis_index("c")
    base = wid * TPW
    ci0 = pltpu.async_copy(gs1_hbm.at[wid], i1v, sem_g)
    ci1 = pltpu.async_copy(gs2_hbm.at[wid], i2v, sem_g)
    cg0 = pltpu.async_copy(g1_hbm.at[pl.ds(base, TPW)], g1v, sem_g)
    cg1 = pltpu.async_copy(g2_hbm.at[pl.ds(base, TPW)], g2v, sem_g)
    ci0.wait()
    ci1.wait()
    cg0.wait()
    cg1.wait()

    vbufs = ((v1a, v2a), (v1b, v2b))
    obufs = (ova, ovb)

    def start(c):
        v1, v2 = vbufs[c % 2]
        a = pltpu.async_copy(eo_hbm.at[i1v.at[c]], v1, sem_g)
        b = pltpu.async_copy(eo_hbm.at[i2v.at[c]], v2, sem_g)
        return a, b

    pend = start(0)
    owrites = [None, None]
    for c in range(_NCH):
        v1, v2 = vbufs[c % 2]
        ov = obufs[c % 2]
        pend[0].wait()
        pend[1].wait()
        if c + 1 < _NCH:
            pend = start(c + 1)
        if owrites[c % 2] is not None:
            owrites[c % 2].wait()

        def row(r, _):
            gv1 = g1v[c * _CH + r]
            gv2 = g2v[c * _CH + r]
            for j in range(D // 16):
                a = v1[r, pl.ds(j * 16, 16)]
                b = v2[r, pl.ds(j * 16, 16)]
                ov[r, pl.ds(j * 16, 16)] = gv1 * a + gv2 * b
            return _

        lax.fori_loop(0, _CH, row, 0)
        owrites[c % 2] = pltpu.async_copy(
            ov, out_hbm.at[pl.ds(base + c * _CH, _CH)], sem_o)
    owrites[0].wait()
    owrites[1].wait()


def _combine_call(eo, gs1, gs2, g1, g2):
    mesh = plsc.VectorSubcoreMesh(core_axis_name="c", subcore_axis_name="s")
    f32 = jnp.float32
    return pl.kernel(
        _combine_body,
        out_type=jax.ShapeDtypeStruct((T, D), f32),
        mesh=mesh,
        scratch_types=[
            pltpu.VMEM((_NCH, _CH), jnp.int32),
            pltpu.VMEM((_NCH, _CH), jnp.int32),
            pltpu.VMEM((TPW, E), f32),
            pltpu.VMEM((TPW, E), f32),
            pltpu.VMEM((_CH, D), f32),
            pltpu.VMEM((_CH, D), f32),
            pltpu.VMEM((_CH, D), f32),
            pltpu.VMEM((_CH, D), f32),
            pltpu.VMEM((_CH, D), f32),
            pltpu.VMEM((_CH, D), f32),
            pltpu.SemaphoreType.DMA,
            pltpu.SemaphoreType.DMA,
        ],
    )(eo, gs1.reshape(NW, _NCH, _CH), gs2.reshape(NW, _NCH, _CH), g1, g2)


def kernel(x, Wg, W1, W2):
    s1, s2, gs1, gs2, g1, g2 = _gating_call(x, Wg)
    s1 = s1.reshape(T)
    s2 = s2.reshape(T)
    gs1 = gs1.reshape(T)
    gs2 = gs2.reshape(T)
    disp = _dispatch_call(x, s1, s2)
    eo = _ffn_call(disp, W1, W2)
    return _combine_call(eo, gs1, gs2, g1, g2)
